# Initial kernel scaffold; baseline (speedup 1.0000x reference)
import jax
import jax.numpy as jnp
from jax.experimental import pallas as pl


def kernel(left, right, X, adj, fc1_W, fc1_b, W1, b1, W2, b2, W3, b3, W4, b4, W5, b5, W6, b6, cnn_W, cnn_b, fcA_W, fcA_b, fcB_W, fcB_b):
    raise NotImplementedError("write your pallas kernel here")


# speedup vs baseline: 795.5528x; 795.5528x over previous
import jax
import jax.numpy as jnp
from jax import lax
from jax.experimental import pallas as pl

NN = 1546
NP = 1664
FH = 256
FG = 128
RB = 128
PB = 512
PREC = lax.Precision.HIGHEST


def _fc1_body(x_ref, w_ref, b_ref, o_ref):
    o_ref[:] = jax.nn.relu(
        jnp.dot(x_ref[:], w_ref[:], precision=PREC) + b_ref[:])


def _build_body(at_ref, atf_ref, aat_ref, abt_ref, dega_ref, degb_ref):
    i = pl.program_id(0)
    at = at_ref[:]
    a2t = jnp.dot(at, atf_ref[:], precision=PREC)
    r = i * RB + lax.broadcasted_iota(jnp.int32, (RB, NP), 0)
    c = lax.broadcasted_iota(jnp.int32, (RB, NP), 1)
    eye = jnp.where((r == c) & (r < NN), 1.0, 0.0).astype(jnp.float32)
    aat = jnp.where(at > 0.5, at, 0.0) + eye
    abt = jnp.where(at + a2t > 0.5, at, 0.0) + eye
    aat_ref[:] = aat
    abt_ref[:] = abt
    dega_ref[:] = jnp.sum(aat, axis=1, keepdims=True)
    degb_ref[:] = jnp.sum(abt, axis=1, keepdims=True)


def _gcn_body(atm_ref, deg_ref, h_ref, w_ref, b_ref, o_ref):
    deg = deg_ref[:]
    dinv = jnp.where(deg > 0.0, lax.rsqrt(deg), 0.0)
    y = jnp.dot(h_ref[:], w_ref[:], precision=PREC) * dinv
    t = jnp.dot(atm_ref[:], y, precision=PREC)
    o_ref[:] = jax.nn.relu(t * dinv + b_ref[:])


def _conv_body(e0_ref, e1_ref, e2_ref, cw_ref, cb_ref, wa1_ref, wa2_ref,
               o_ref, u_ref, v_ref):
    acc = jnp.zeros((NN, FH), jnp.float32)
    for idx, eref in enumerate((e0_ref, e1_ref, e2_ref)):
        x = eref[:]
        for kh in range(3):
            for kw in range(3):
                w = cw_ref[0, idx * 9 + kh * 3 + kw]
                acc = acc + w * lax.slice(
                    x, (2 * kh, 2 * kw), (2 * kh + NN, 2 * kw + FH))
    acc = acc + cb_ref[0, 0]
    o_ref[:] = acc
    u_ref[:] = jnp.dot(acc, wa1_ref[:], precision=PREC)
    v_ref[:] = jnp.dot(acc, wa2_ref[:], precision=PREC)


def _pair_body(l_ref, r_ref, u_ref, v_ref, ab_ref, wb_ref, bb_ref, o_ref):
    c = lax.broadcasted_iota(jnp.int32, (PB, NP), 1)
    oh_l = (l_ref[:] == c).astype(jnp.float32)
    oh_r = (r_ref[:] == c).astype(jnp.float32)
    s = (jnp.dot(oh_l, u_ref[:], precision=PREC)
         + jnp.dot(oh_r, v_ref[:], precision=PREC))
    h = jax.nn.relu(s + ab_ref[:])
    o_ref[:] = jnp.dot(h, wb_ref[:], precision=PREC) + bb_ref[:]


def kernel(left, right, X, adj, fc1_W, fc1_b, W1, b1, W2, b2, W3, b3,
           W4, b4, W5, b5, W6, b6, cnn_W, cnn_b, fcA_W, fcA_b, fcB_W, fcB_b):
    f32 = jnp.float32
    pad_n = NP - NN
    Xp = jnp.pad(X, ((0, pad_n), (0, pad_n)))
    adjTp = jnp.pad(adj.T, ((0, pad_n), (0, pad_n)))
    fc1_Wp = jnp.pad(fc1_W, ((0, pad_n), (0, 0)))

    Xf = pl.pallas_call(
        _fc1_body,
        out_shape=jax.ShapeDtypeStruct((NP, FH), f32),
    )(Xp, fc1_Wp, fc1_b.reshape(1, FH))

    nblk = NP // RB
    AaT, AbT, dega, degb = pl.pallas_call(
        _build_body,
        grid=(nblk,),
        in_specs=[
            pl.BlockSpec((RB, NP), lambda i: (i, 0)),
            pl.BlockSpec((NP, NP), lambda i: (0, 0)),
        ],
        out_specs=[
            pl.BlockSpec((RB, NP), lambda i: (i, 0)),
            pl.BlockSpec((RB, NP), lambda i: (i, 0)),
            pl.BlockSpec((RB, 1), lambda i: (i, 0)),
            pl.BlockSpec((RB, 1), lambda i: (i, 0)),
        ],
        out_shape=[
            jax.ShapeDtypeStruct((NP, NP), f32),
            jax.ShapeDtypeStruct((NP, NP), f32),
            jax.ShapeDtypeStruct((NP, 1), f32),
            jax.ShapeDtypeStruct((NP, 1), f32),
        ],
    )(adjTp, adjTp)

    def gcn(atm, deg, h, w, b):
        return pl.pallas_call(
            _gcn_body,
            out_shape=jax.ShapeDtypeStruct((NP, FG), f32),
        )(atm, deg, h, w, b.reshape(1, FG))

    e1 = gcn(AbT, degb, Xf, W1, b1)
    e2 = gcn(AbT, degb, e1, W2, b2)
    e3 = gcn(AaT, dega, Xf, W3, b3)
    e4 = gcn(AaT, dega, e3, W4, b4)
    e5 = gcn(AaT, dega, Xf, W5, b5)
    e6 = gcn(AaT, dega, e5, W6, b6)

    def branch(eo, ee):
        return jnp.pad(
            jnp.concatenate([eo[:NN], ee[:NN]], axis=1), ((2, 2), (2, 2)))

    ep0 = branch(e5, e6)
    ep1 = branch(e3, e4)
    ep2 = branch(e1, e2)

    emb_all, U, V = pl.pallas_call(
        _conv_body,
        out_shape=[
            jax.ShapeDtypeStruct((NN, FH), f32),
            jax.ShapeDtypeStruct((NN, 64), f32),
            jax.ShapeDtypeStruct((NN, 64), f32),
        ],
    )(ep0, ep1, ep2, cnn_W.reshape(1, 27), cnn_b.reshape(1, 1),
      fcA_W[:FH], fcA_W[FH:])

    Bsz = left.shape[0]
    Up = jnp.pad(U, ((0, pad_n), (0, 0)))
    Vp = jnp.pad(V, ((0, pad_n), (0, 0)))
    wbp = jnp.pad(fcB_W, ((0, 0), (0, 128 - fcB_W.shape[1])))
    bbp = jnp.pad(fcB_b.reshape(1, -1), ((0, 0), (0, 128 - fcB_b.shape[0])))

    outp = pl.pallas_call(
        _pair_body,
        grid=(Bsz // PB,),
        in_specs=[
            pl.BlockSpec((PB, 1), lambda i: (i, 0)),
            pl.BlockSpec((PB, 1), lambda i: (i, 0)),
            pl.BlockSpec((NP, 64), lambda i: (0, 0)),
            pl.BlockSpec((NP, 64), lambda i: (0, 0)),
            pl.BlockSpec((1, 64), lambda i: (0, 0)),
            pl.BlockSpec((64, 128), lambda i: (0, 0)),
            pl.BlockSpec((1, 128), lambda i: (0, 0)),
        ],
        out_specs=pl.BlockSpec((PB, 128), lambda i: (i, 0)),
        out_shape=jax.ShapeDtypeStruct((Bsz, 128), f32),
    )(left.reshape(Bsz, 1), right.reshape(Bsz, 1), Up, Vp,
      fcA_b.reshape(1, 64), wbp, bbp)

    return outp[:, :2], emb_all


# speedup vs baseline: 963.0625x; 1.2106x over previous
import jax
import jax.numpy as jnp
from jax import lax
from jax.experimental import pallas as pl

NN = 1546
NP = 1664
FH = 256
FG = 128
RB = 128
PB = 512
PREC = lax.Precision.HIGHEST
MATCH = lax.Precision.DEFAULT


def _fc1_body(x_ref, w_ref, b_ref, o_ref):
    o_ref[:] = jax.nn.relu(
        jnp.dot(x_ref[:], w_ref[:], precision=MATCH) + b_ref[:])


def _build_body(at_ref, atf_ref, aat_ref, abt_ref, dega_ref, degb_ref):
    i = pl.program_id(0)
    at = at_ref[:]
    a2t = jnp.dot(at, atf_ref[:], precision=MATCH)
    r = i * RB + lax.broadcasted_iota(jnp.int32, (RB, NP), 0)
    c = lax.broadcasted_iota(jnp.int32, (RB, NP), 1)
    eye = jnp.where((r == c) & (r < NN), 1.0, 0.0).astype(jnp.float32)
    aat = jnp.where(at > 0.5, at, 0.0) + eye
    abt = jnp.where(at + a2t > 0.5, at, 0.0) + eye
    aat_ref[:] = aat
    abt_ref[:] = abt
    dega_ref[:] = jnp.sum(aat, axis=1, keepdims=True)
    degb_ref[:] = jnp.sum(abt, axis=1, keepdims=True)


def _gcn_body(atm_ref, deg_ref, h_ref, w_ref, b_ref, o_ref):
    deg = deg_ref[:]
    dinv = jnp.where(deg > 0.0, lax.rsqrt(deg), 0.0)
    y = jnp.dot(h_ref[:], w_ref[:], precision=MATCH) * dinv
    t = jnp.dot(atm_ref[:], y, precision=PREC)
    o_ref[:] = jax.nn.relu(t * dinv + b_ref[:])


def _conv_body(e0_ref, e1_ref, e2_ref, cw_ref, cb_ref, wa1_ref, wa2_ref,
               o_ref, u_ref, v_ref):
    bf16, f32 = jnp.bfloat16, jnp.float32
    acc = jnp.zeros((NN, FH), jnp.float32)
    for idx, eref in enumerate((e0_ref, e1_ref, e2_ref)):
        x = eref[:].astype(bf16).astype(f32)
        for kh in range(3):
            for kw in range(3):
                w = cw_ref[0, idx * 9 + kh * 3 + kw].astype(bf16).astype(f32)
                acc = acc + w * lax.slice(
                    x, (2 * kh, 2 * kw), (2 * kh + NN, 2 * kw + FH))
    acc = acc + cb_ref[0, 0]
    o_ref[:] = acc
    u_ref[:] = jnp.dot(acc, wa1_ref[:], precision=MATCH)
    v_ref[:] = jnp.dot(acc, wa2_ref[:], precision=MATCH)


def _pair_body(l_ref, r_ref, u_ref, v_ref, ab_ref, wb_ref, bb_ref, o_ref):
    c = lax.broadcasted_iota(jnp.int32, (PB, NP), 1)
    oh_l = (l_ref[:] == c).astype(jnp.float32)
    oh_r = (r_ref[:] == c).astype(jnp.float32)
    s = (jnp.dot(oh_l, u_ref[:], precision=PREC)
         + jnp.dot(oh_r, v_ref[:], precision=PREC))
    h = jax.nn.relu(s + ab_ref[:])
    o_ref[:] = jnp.dot(h, wb_ref[:], precision=MATCH) + bb_ref[:]


def kernel(left, right, X, adj, fc1_W, fc1_b, W1, b1, W2, b2, W3, b3,
           W4, b4, W5, b5, W6, b6, cnn_W, cnn_b, fcA_W, fcA_b, fcB_W, fcB_b):
    f32 = jnp.float32
    pad_n = NP - NN
    Xp = jnp.pad(X, ((0, pad_n), (0, pad_n)))
    adjTp = jnp.pad(adj.T, ((0, pad_n), (0, pad_n)))
    fc1_Wp = jnp.pad(fc1_W, ((0, pad_n), (0, 0)))

    Xf = pl.pallas_call(
        _fc1_body,
        out_shape=jax.ShapeDtypeStruct((NP, FH), f32),
    )(Xp, fc1_Wp, fc1_b.reshape(1, FH))

    nblk = NP // RB
    AaT, AbT, dega, degb = pl.pallas_call(
        _build_body,
        grid=(nblk,),
        in_specs=[
            pl.BlockSpec((RB, NP), lambda i: (i, 0)),
            pl.BlockSpec((NP, NP), lambda i: (0, 0)),
        ],
        out_specs=[
            pl.BlockSpec((RB, NP), lambda i: (i, 0)),
            pl.BlockSpec((RB, NP), lambda i: (i, 0)),
            pl.BlockSpec((RB, 1), lambda i: (i, 0)),
            pl.BlockSpec((RB, 1), lambda i: (i, 0)),
        ],
        out_shape=[
            jax.ShapeDtypeStruct((NP, NP), f32),
            jax.ShapeDtypeStruct((NP, NP), f32),
            jax.ShapeDtypeStruct((NP, 1), f32),
            jax.ShapeDtypeStruct((NP, 1), f32),
        ],
    )(adjTp, adjTp)

    def gcn(atm, deg, h, w, b):
        return pl.pallas_call(
            _gcn_body,
            out_shape=jax.ShapeDtypeStruct((NP, FG), f32),
        )(atm, deg, h, w, b.reshape(1, FG))

    e1 = gcn(AbT, degb, Xf, W1, b1)
    e2 = gcn(AbT, degb, e1, W2, b2)
    e3 = gcn(AaT, dega, Xf, W3, b3)
    e4 = gcn(AaT, dega, e3, W4, b4)
    e5 = gcn(AaT, dega, Xf, W5, b5)
    e6 = gcn(AaT, dega, e5, W6, b6)

    def branch(eo, ee):
        return jnp.pad(
            jnp.concatenate([eo[:NN], ee[:NN]], axis=1), ((2, 2), (2, 2)))

    ep0 = branch(e5, e6)
    ep1 = branch(e3, e4)
    ep2 = branch(e1, e2)

    emb_all, U, V = pl.pallas_call(
        _conv_body,
        out_shape=[
            jax.ShapeDtypeStruct((NN, FH), f32),
            jax.ShapeDtypeStruct((NN, 64), f32),
            jax.ShapeDtypeStruct((NN, 64), f32),
        ],
    )(ep0, ep1, ep2, cnn_W.reshape(1, 27), cnn_b.reshape(1, 1),
      fcA_W[:FH], fcA_W[FH:])

    Bsz = left.shape[0]
    Up = jnp.pad(U, ((0, pad_n), (0, 0)))
    Vp = jnp.pad(V, ((0, pad_n), (0, 0)))
    wbp = jnp.pad(fcB_W, ((0, 0), (0, 128 - fcB_W.shape[1])))
    bbp = jnp.pad(fcB_b.reshape(1, -1), ((0, 0), (0, 128 - fcB_b.shape[0])))

    outp = pl.pallas_call(
        _pair_body,
        grid=(Bsz // PB,),
        in_specs=[
            pl.BlockSpec((PB, 1), lambda i: (i, 0)),
            pl.BlockSpec((PB, 1), lambda i: (i, 0)),
            pl.BlockSpec((NP, 64), lambda i: (0, 0)),
            pl.BlockSpec((NP, 64), lambda i: (0, 0)),
            pl.BlockSpec((1, 64), lambda i: (0, 0)),
            pl.BlockSpec((64, 128), lambda i: (0, 0)),
            pl.BlockSpec((1, 128), lambda i: (0, 0)),
        ],
        out_specs=pl.BlockSpec((PB, 128), lambda i: (i, 0)),
        out_shape=jax.ShapeDtypeStruct((Bsz, 128), f32),
    )(left.reshape(Bsz, 1), right.reshape(Bsz, 1), Up, Vp,
      fcA_b.reshape(1, 64), wbp, bbp)

    return outp[:, :2], emb_all


# speedup vs baseline: 1482.7947x; 1.5397x over previous
import jax
import jax.numpy as jnp
from jax import lax
from jax.experimental import pallas as pl
from jax.experimental.pallas import tpu as pltpu
from jax.experimental.pallas import tpu_sc as plsc

NN = 1546
NP = 1664
FH = 256
FG = 128
RB = 128
PB = 512
PREC = lax.Precision.HIGHEST
MATCH = lax.Precision.DEFAULT


def _fc1_body(x_ref, w_ref, b_ref, o_ref):
    o_ref[:] = jax.nn.relu(
        jnp.dot(x_ref[:], w_ref[:], precision=MATCH) + b_ref[:])


def _build_body(at_ref, atf_ref, aat_ref, abt_ref, dega_ref, degb_ref):
    i = pl.program_id(0)
    at = at_ref[:]
    a2t = jnp.dot(at, atf_ref[:], precision=MATCH)
    r = i * RB + lax.broadcasted_iota(jnp.int32, (RB, NP), 0)
    c = lax.broadcasted_iota(jnp.int32, (RB, NP), 1)
    eye = jnp.where((r == c) & (r < NN), 1.0, 0.0).astype(jnp.float32)
    aat = jnp.where(at > 0.5, at, 0.0) + eye
    abt = jnp.where(at + a2t > 0.5, at, 0.0) + eye
    aat_ref[:] = aat
    abt_ref[:] = abt
    dega_ref[:] = jnp.sum(aat, axis=1, keepdims=True)
    degb_ref[:] = jnp.sum(abt, axis=1, keepdims=True)


def _gcn_body(atm_ref, deg_ref, h_ref, w_ref, b_ref, o_ref):
    deg = deg_ref[:]
    dinv = jnp.where(deg > 0.0, lax.rsqrt(deg), 0.0)
    y = jnp.dot(h_ref[:], w_ref[:], precision=MATCH) * dinv
    t = jnp.dot(atm_ref[:], y, precision=PREC)
    o_ref[:] = jax.nn.relu(t * dinv + b_ref[:])


def _conv_body(e0_ref, e1_ref, e2_ref, cw_ref, cb_ref, wa1_ref, wa2_ref,
               o_ref, uv_ref):
    bf16, f32 = jnp.bfloat16, jnp.float32
    acc = jnp.zeros((NN, FH), jnp.float32)
    for idx, eref in enumerate((e0_ref, e1_ref, e2_ref)):
        x = eref[:].astype(bf16).astype(f32)
        for kh in range(3):
            for kw in range(3):
                w = cw_ref[0, idx * 9 + kh * 3 + kw].astype(bf16).astype(f32)
                acc = acc + w * lax.slice(
                    x, (2 * kh, 2 * kw), (2 * kh + NN, 2 * kw + FH))
    acc = acc + cb_ref[0, 0]
    o_ref[:] = acc
    uv_ref[:, :64] = jnp.dot(acc, wa1_ref[:], precision=MATCH)
    uv_ref[:, 64:] = jnp.dot(acc, wa2_ref[:], precision=MATCH)


GW = 128
_VMESH = plsc.VectorSubcoreMesh(core_axis_name="c", subcore_axis_name="s")


def _pair_gather(T, li, ri):
    B = li.shape[1]
    f32 = jnp.float32

    def body(t_hbm, l_hbm, r_hbm, gl_hbm, gr_hbm):
        def inner(l_vmem, r_vmem, gl_vmem, gr_vmem):
            pltpu.sync_copy(t_hbm.at[l_vmem.at[0]], gl_vmem)
            pltpu.sync_copy(t_hbm.at[r_vmem.at[0]], gr_vmem)

        pltpu.emit_pipeline(
            inner,
            grid=(B // GW,),
            in_specs=[pl.BlockSpec((1, GW), lambda i: (0, i)),
                      pl.BlockSpec((1, GW), lambda i: (0, i))],
            out_specs=[pl.BlockSpec((GW, 128), lambda i: (i, 0)),
                       pl.BlockSpec((GW, 128), lambda i: (i, 0))],
            core_axis_name=("c", "s"),
            dimension_semantics=(pltpu.PARALLEL,),
        )(l_hbm, r_hbm, gl_hbm, gr_hbm)

    k = pl.kernel(body,
                  out_type=[jax.ShapeDtypeStruct((B, 128), f32),
                            jax.ShapeDtypeStruct((B, 128), f32)],
                  mesh=_VMESH)
    return k(T, li, ri)


def _pairfin_body(gl_ref, gr_ref, ab_ref, wb_ref, bb_ref, o_ref):
    h = jax.nn.relu(gl_ref[:, :64] + gr_ref[:, 64:] + ab_ref[:])
    o_ref[:] = jnp.dot(h, wb_ref[:], precision=MATCH) + bb_ref[:]


def kernel(left, right, X, adj, fc1_W, fc1_b, W1, b1, W2, b2, W3, b3,
           W4, b4, W5, b5, W6, b6, cnn_W, cnn_b, fcA_W, fcA_b, fcB_W, fcB_b):
    f32 = jnp.float32
    pad_n = NP - NN
    Xp = jnp.pad(X, ((0, pad_n), (0, pad_n)))
    adjTp = jnp.pad(adj.T, ((0, pad_n), (0, pad_n)))
    fc1_Wp = jnp.pad(fc1_W, ((0, pad_n), (0, 0)))

    Xf = pl.pallas_call(
        _fc1_body,
        out_shape=jax.ShapeDtypeStruct((NP, FH), f32),
    )(Xp, fc1_Wp, fc1_b.reshape(1, FH))

    nblk = NP // RB
    AaT, AbT, dega, degb = pl.pallas_call(
        _build_body,
        grid=(nblk,),
        in_specs=[
            pl.BlockSpec((RB, NP), lambda i: (i, 0)),
            pl.BlockSpec((NP, NP), lambda i: (0, 0)),
        ],
        out_specs=[
            pl.BlockSpec((RB, NP), lambda i: (i, 0)),
            pl.BlockSpec((RB, NP), lambda i: (i, 0)),
            pl.BlockSpec((RB, 1), lambda i: (i, 0)),
            pl.BlockSpec((RB, 1), lambda i: (i, 0)),
        ],
        out_shape=[
            jax.ShapeDtypeStruct((NP, NP), f32),
            jax.ShapeDtypeStruct((NP, NP), f32),
            jax.ShapeDtypeStruct((NP, 1), f32),
            jax.ShapeDtypeStruct((NP, 1), f32),
        ],
    )(adjTp, adjTp)

    def gcn(atm, deg, h, w, b):
        return pl.pallas_call(
            _gcn_body,
            out_shape=jax.ShapeDtypeStruct((NP, FG), f32),
        )(atm, deg, h, w, b.reshape(1, FG))

    e1 = gcn(AbT, degb, Xf, W1, b1)
    e2 = gcn(AbT, degb, e1, W2, b2)
    e3 = gcn(AaT, dega, Xf, W3, b3)
    e4 = gcn(AaT, dega, e3, W4, b4)
    e5 = gcn(AaT, dega, Xf, W5, b5)
    e6 = gcn(AaT, dega, e5, W6, b6)

    def branch(eo, ee):
        return jnp.pad(
            jnp.concatenate([eo[:NN], ee[:NN]], axis=1), ((2, 2), (2, 2)))

    ep0 = branch(e5, e6)
    ep1 = branch(e3, e4)
    ep2 = branch(e1, e2)

    emb_all, UV = pl.pallas_call(
        _conv_body,
        out_shape=[
            jax.ShapeDtypeStruct((NN, FH), f32),
            jax.ShapeDtypeStruct((NN, 128), f32),
        ],
    )(ep0, ep1, ep2, cnn_W.reshape(1, 27), cnn_b.reshape(1, 1),
      fcA_W[:FH], fcA_W[FH:])

    Bsz = left.shape[0]
    GL, GR = _pair_gather(UV, left.reshape(1, Bsz), right.reshape(1, Bsz))

    wbp = jnp.pad(fcB_W, ((0, 0), (0, 128 - fcB_W.shape[1])))
    bbp = jnp.pad(fcB_b.reshape(1, -1), ((0, 0), (0, 128 - fcB_b.shape[0])))

    outp = pl.pallas_call(
        _pairfin_body,
        grid=(Bsz // PB,),
        in_specs=[
            pl.BlockSpec((PB, 128), lambda i: (i, 0)),
            pl.BlockSpec((PB, 128), lambda i: (i, 0)),
            pl.BlockSpec((1, 64), lambda i: (0, 0)),
            pl.BlockSpec((64, 128), lambda i: (0, 0)),
            pl.BlockSpec((1, 128), lambda i: (0, 0)),
        ],
        out_specs=pl.BlockSpec((PB, 128), lambda i: (i, 0)),
        out_shape=jax.ShapeDtypeStruct((Bsz, 128), f32),
    )(GL, GR, fcA_b.reshape(1, 64), wbp, bbp)

    return outp[:, :2], emb_all


# speedup vs baseline: 1804.4754x; 1.2169x over previous
import jax
import jax.numpy as jnp
from jax import lax
from jax.experimental import pallas as pl
from jax.experimental.pallas import tpu as pltpu
from jax.experimental.pallas import tpu_sc as plsc

NN = 1546
FH = 256
FG = 128
RB = 128
PB = 512
GW = 128
PREC = lax.Precision.HIGHEST
MATCH = lax.Precision.DEFAULT

_DN_T = (((0,), (0,)), ((), ()))


def _fc1_body(x_ref, w_ref, b_ref, o_ref):
    o_ref[:] = jax.nn.relu(
        jnp.dot(x_ref[:], w_ref[:], precision=MATCH) + b_ref[:])


def _build_body(a_ref, af_ref, aa_ref, ab_ref, dega_ref, degb_ref):
    i = pl.program_id(0)
    a = a_ref[:]
    a2 = jnp.dot(a, af_ref[:], precision=MATCH)
    r = i * RB + lax.broadcasted_iota(jnp.int32, (RB, NN), 0)
    c = lax.broadcasted_iota(jnp.int32, (RB, NN), 1)
    valid = r < NN
    eye = jnp.where((r == c) & valid, 1.0, 0.0).astype(jnp.float32)
    aa = jnp.where(valid & (a > 0.5), a, 0.0) + eye
    ab = jnp.where(valid & (a + a2 > 0.5), a, 0.0) + eye
    aa_ref[:] = aa
    ab_ref[:] = ab
    ones = jnp.ones((RB, 1), jnp.float32)
    da = lax.dot_general(aa, ones, _DN_T, precision=PREC)
    db = lax.dot_general(ab, ones, _DN_T, precision=PREC)

    @pl.when(i == 0)
    def _():
        dega_ref[:] = da
        degb_ref[:] = db

    @pl.when(i > 0)
    def _():
        dega_ref[:] += da
        degb_ref[:] += db


def _gcn(am_ref, dinv, h, w_ref, b_ref):
    y = jnp.dot(h, w_ref[:], precision=MATCH) * dinv
    t = lax.dot_general(am_ref[:], y, _DN_T, precision=PREC)
    return jax.nn.relu(t * dinv + b_ref[:])


def _gcnb_body(ab_ref, degb_ref, xf_ref, w1_ref, b1_ref, w2_ref, b2_ref,
               e1_ref, e2_ref):
    dib = jnp.where(degb_ref[:] > 0.0, lax.rsqrt(degb_ref[:]), 0.0)
    e1 = _gcn(ab_ref, dib, xf_ref[:], w1_ref, b1_ref)
    e1_ref[:] = e1
    e2_ref[:] = _gcn(ab_ref, dib, e1, w2_ref, b2_ref)


def _gcna_body(aa_ref, dega_ref, xf_ref, w3_ref, b3_ref, w4_ref, b4_ref,
               w5_ref, b5_ref, w6_ref, b6_ref,
               e3_ref, e4_ref, e5_ref, e6_ref):
    dia = jnp.where(dega_ref[:] > 0.0, lax.rsqrt(dega_ref[:]), 0.0)
    xf = xf_ref[:]
    e3 = _gcn(aa_ref, dia, xf, w3_ref, b3_ref)
    e3_ref[:] = e3
    e4_ref[:] = _gcn(aa_ref, dia, e3, w4_ref, b4_ref)
    e5 = _gcn(aa_ref, dia, xf, w5_ref, b5_ref)
    e5_ref[:] = e5
    e6_ref[:] = _gcn(aa_ref, dia, e5, w6_ref, b6_ref)


def _conv_body(e1_ref, e2_ref, e3_ref, e4_ref, e5_ref, e6_ref,
               cw_ref, cb_ref, wa1_ref, wa2_ref, o_ref, uv_ref):
    bf16, f32 = jnp.bfloat16, jnp.float32
    zr = jnp.zeros((2, FH + 4), f32)
    zc = jnp.zeros((NN, 2), f32)
    acc = jnp.zeros((NN, FH), f32)
    for idx, (eo_ref, ee_ref) in enumerate(
            ((e5_ref, e6_ref), (e3_ref, e4_ref), (e1_ref, e2_ref))):
        x = jnp.concatenate([zc, eo_ref[:], ee_ref[:], zc], axis=1)
        x = jnp.concatenate([zr, x, zr], axis=0)
        x = x.astype(bf16).astype(f32)
        for kh in range(3):
            for kw in range(3):
                w = cw_ref[0, idx * 9 + kh * 3 + kw].astype(bf16).astype(f32)
                acc = acc + w * lax.slice(
                    x, (2 * kh, 2 * kw), (2 * kh + NN, 2 * kw + FH))
    acc = acc + cb_ref[0, 0]
    o_ref[:] = acc
    uv_ref[:, :64] = jnp.dot(acc, wa1_ref[:], precision=MATCH)
    uv_ref[:, 64:] = jnp.dot(acc, wa2_ref[:], precision=MATCH)


_VMESH = plsc.VectorSubcoreMesh(core_axis_name="c", subcore_axis_name="s")


def _pair_gather(T, li, ri):
    B = li.shape[1]
    f32 = jnp.float32

    def body(t_hbm, l_hbm, r_hbm, gl_hbm, gr_hbm):
        def inner(l_vmem, r_vmem, gl_vmem, gr_vmem):
            pltpu.sync_copy(t_hbm.at[l_vmem.at[0]], gl_vmem)
            pltpu.sync_copy(t_hbm.at[r_vmem.at[0]], gr_vmem)

        pltpu.emit_pipeline(
            inner,
            grid=(B // GW,),
            in_specs=[pl.BlockSpec((1, GW), lambda i: (0, i)),
                      pl.BlockSpec((1, GW), lambda i: (0, i))],
            out_specs=[pl.BlockSpec((GW, 128), lambda i: (i, 0)),
                       pl.BlockSpec((GW, 128), lambda i: (i, 0))],
            core_axis_name=("c", "s"),
            dimension_semantics=(pltpu.PARALLEL,),
        )(l_hbm, r_hbm, gl_hbm, gr_hbm)

    k = pl.kernel(body,
                  out_type=[jax.ShapeDtypeStruct((B, 128), f32),
                            jax.ShapeDtypeStruct((B, 128), f32)],
                  mesh=_VMESH)
    return k(T, li, ri)


def _pairfin_body(gl_ref, gr_ref, ab_ref, wb_ref, bb_ref, o_ref):
    h = jax.nn.relu(gl_ref[:, :64] + gr_ref[:, 64:] + ab_ref[:])
    o_ref[:] = jnp.dot(h, wb_ref[:], precision=MATCH) + bb_ref[:]


def kernel(left, right, X, adj, fc1_W, fc1_b, W1, b1, W2, b2, W3, b3,
           W4, b4, W5, b5, W6, b6, cnn_W, cnn_b, fcA_W, fcA_b, fcB_W, fcB_b):
    f32 = jnp.float32

    Xf = pl.pallas_call(
        _fc1_body,
        out_shape=jax.ShapeDtypeStruct((NN, FH), f32),
    )(X, fc1_W, fc1_b.reshape(1, FH))

    nblk = pl.cdiv(NN, RB)
    Aa, Ab, dega, degb = pl.pallas_call(
        _build_body,
        grid=(nblk,),
        in_specs=[
            pl.BlockSpec((RB, NN), lambda i: (i, 0)),
            pl.BlockSpec((NN, NN), lambda i: (0, 0)),
        ],
        out_specs=[
            pl.BlockSpec((RB, NN), lambda i: (i, 0)),
            pl.BlockSpec((RB, NN), lambda i: (i, 0)),
            pl.BlockSpec((NN, 1), lambda i: (0, 0)),
            pl.BlockSpec((NN, 1), lambda i: (0, 0)),
        ],
        out_shape=[
            jax.ShapeDtypeStruct((NN, NN), f32),
            jax.ShapeDtypeStruct((NN, NN), f32),
            jax.ShapeDtypeStruct((NN, 1), f32),
            jax.ShapeDtypeStruct((NN, 1), f32),
        ],
    )(adj, adj)

    eshape = jax.ShapeDtypeStruct((NN, FG), f32)
    e1, e2 = pl.pallas_call(
        _gcnb_body, out_shape=[eshape, eshape],
    )(Ab, degb, Xf, W1, b1.reshape(1, FG), W2, b2.reshape(1, FG))
    e3, e4, e5, e6 = pl.pallas_call(
        _gcna_body, out_shape=[eshape, eshape, eshape, eshape],
    )(Aa, dega, Xf, W3, b3.reshape(1, FG), W4, b4.reshape(1, FG),
      W5, b5.reshape(1, FG), W6, b6.reshape(1, FG))

    emb_all, UV = pl.pallas_call(
        _conv_body,
        out_shape=[
            jax.ShapeDtypeStruct((NN, FH), f32),
            jax.ShapeDtypeStruct((NN, 128), f32),
        ],
    )(e1, e2, e3, e4, e5, e6,
      cnn_W.reshape(1, 27), cnn_b.reshape(1, 1), fcA_W[:FH], fcA_W[FH:])

    Bsz = left.shape[0]
    GL, GR = _pair_gather(UV, left.reshape(1, Bsz), right.reshape(1, Bsz))

    wbp = jnp.pad(fcB_W, ((0, 0), (0, 128 - fcB_W.shape[1])))
    bbp = jnp.pad(fcB_b.reshape(1, -1), ((0, 0), (0, 128 - fcB_b.shape[0])))

    outp = pl.pallas_call(
        _pairfin_body,
        grid=(Bsz // PB,),
        in_specs=[
            pl.BlockSpec((PB, 128), lambda i: (i, 0)),
            pl.BlockSpec((PB, 128), lambda i: (i, 0)),
            pl.BlockSpec((1, 64), lambda i: (0, 0)),
            pl.BlockSpec((64, 128), lambda i: (0, 0)),
            pl.BlockSpec((1, 128), lambda i: (0, 0)),
        ],
        out_specs=pl.BlockSpec((PB, 128), lambda i: (i, 0)),
        out_shape=jax.ShapeDtypeStruct((Bsz, 128), f32),
    )(GL, GR, fcA_b.reshape(1, 64), wbp, bbp)

    return outp[:, :2], emb_all
